# Initial kernel scaffold; baseline (speedup 1.0000x reference)
import jax
import jax.numpy as jnp
from jax.experimental import pallas as pl


def kernel(z, weight):
    raise NotImplementedError("write your pallas kernel here")


# speedup vs baseline: 1.0866x; 1.0866x over previous
import functools

import jax
import jax.numpy as jnp
from jax import lax
from jax.experimental import pallas as pl
from jax.experimental.pallas import tpu as pltpu

NUM_CODES = 8192
DIM = 256
TOKENS = 8192
BM = 512
NB = TOKENS // BM


def _dist_argmin_kernel(z_ref, w_ref, idx_ref, dmin_ref):
    z = z_ref[...]
    w = w_ref[...]
    zsq = jnp.sum(z * z, axis=1, keepdims=True)
    csq = jnp.sum(w * w, axis=1)
    mm = lax.dot_general(z, w, (((1,), (1,)), ((), ())),
                         preferred_element_type=jnp.float32)
    dist = (zsq + csq[None, :]) - 2.0 * mm
    mn = jnp.min(dist, axis=1, keepdims=True)
    ii = lax.broadcasted_iota(jnp.int32, dist.shape, 1)
    idx = jnp.min(jnp.where(dist == mn, ii, jnp.int32(2**30)), axis=1)
    idx_ref[0, 0, :] = idx
    dmin_ref[0, 0, :] = mn[:, 0]


def _dist_argmin(z_flat, weight):
    return pl.pallas_call(
        _dist_argmin_kernel,
        grid=(NB,),
        in_specs=[
            pl.BlockSpec((BM, DIM), lambda i: (i, 0)),
            pl.BlockSpec((NUM_CODES, DIM), lambda i: (0, 0)),
        ],
        out_specs=[
            pl.BlockSpec((1, 1, BM), lambda i: (i, 0, 0)),
            pl.BlockSpec((1, 1, BM), lambda i: (i, 0, 0)),
        ],
        out_shape=[
            jax.ShapeDtypeStruct((NB, 1, BM), jnp.int32),
            jax.ShapeDtypeStruct((NB, 1, BM), jnp.float32),
        ],
        compiler_params=pltpu.CompilerParams(
            dimension_semantics=("arbitrary",),
        ),
    )(z_flat, weight)


def kernel(z, weight):
    zp = jnp.transpose(z, (0, 2, 3, 1))
    b, h, w, d = zp.shape
    z_flat = zp.reshape(-1, d)
    idx3, _ = _dist_argmin(z_flat, weight)
    vq_indices = idx3.reshape(-1)
    z_quantized = jnp.take(weight, vq_indices, axis=0)
    commitment_loss = jnp.mean((z_flat - z_quantized) ** 2)
    loss = 0.25 * commitment_loss
    q = jnp.transpose(z_quantized.reshape(b, h, w, d), (0, 3, 1, 2))
    return (q, loss, commitment_loss)


# speedup vs baseline: 1.1864x; 1.0918x over previous
import functools

import jax
import jax.numpy as jnp
from jax import lax
from jax.experimental import pallas as pl
from jax.experimental.pallas import tpu as pltpu
from jax.experimental.pallas import tpu_sc as plsc

NUM_CODES = 8192
DIM = 256
TOKENS = 8192
BM = 512
NB = TOKENS // BM

_SC_INFO = plsc.get_sparse_core_info()
_NC, _NS = _SC_INFO.num_cores, _SC_INFO.num_subcores
_NW = _NC * _NS
_BPW = TOKENS // _NW


def _dist_argmin_kernel(z_ref, w_ref, idx_ref, dmin_ref):
    z = z_ref[...]
    w = w_ref[...]
    zsq = jnp.sum(z * z, axis=1, keepdims=True)
    csq = jnp.sum(w * w, axis=1)
    mm = lax.dot_general(z, w, (((1,), (1,)), ((), ())),
                         preferred_element_type=jnp.float32)
    dist = (zsq + csq[None, :]) - 2.0 * mm
    mn = jnp.min(dist, axis=1, keepdims=True)
    ii = lax.broadcasted_iota(jnp.int32, dist.shape, 1)
    idx = jnp.min(jnp.where(dist == mn, ii, jnp.int32(2**30)), axis=1)
    idx_ref[0, 0, :] = idx
    dmin_ref[0, 0, :] = mn[:, 0]


def _dist_argmin(z_flat, weight):
    return pl.pallas_call(
        _dist_argmin_kernel,
        grid=(NB,),
        in_specs=[
            pl.BlockSpec((BM, DIM), lambda i: (i, 0)),
            pl.BlockSpec((NUM_CODES, DIM), lambda i: (0, 0)),
        ],
        out_specs=[
            pl.BlockSpec((1, 1, BM), lambda i: (i, 0, 0)),
            pl.BlockSpec((1, 1, BM), lambda i: (i, 0, 0)),
        ],
        out_shape=[
            jax.ShapeDtypeStruct((NB, 1, BM), jnp.int32),
            jax.ShapeDtypeStruct((NB, 1, BM), jnp.float32),
        ],
        compiler_params=pltpu.CompilerParams(
            dimension_semantics=("arbitrary",),
        ),
    )(z_flat, weight)


@functools.partial(
    pl.kernel,
    mesh=plsc.VectorSubcoreMesh(core_axis_name="c", subcore_axis_name="s"),
    out_type=jax.ShapeDtypeStruct((TOKENS, DIM), jnp.float32),
    scratch_types=[
        pltpu.VMEM((_BPW,), jnp.int32),
        pltpu.VMEM((_BPW, DIM), jnp.float32),
        pltpu.SemaphoreType.DMA,
    ],
)
def _sc_gather(table_hbm, idx_hbm, out_hbm, idx_v, rows_v, sem):
    wid = lax.axis_index("s") * _NC + lax.axis_index("c")
    base = wid * _BPW
    pltpu.sync_copy(idx_hbm.at[pl.ds(base, _BPW)], idx_v)
    pltpu.async_copy(table_hbm.at[idx_v], rows_v, sem).wait()
    pltpu.sync_copy(rows_v, out_hbm.at[pl.ds(base, _BPW)])


def kernel(z, weight):
    zp = jnp.transpose(z, (0, 2, 3, 1))
    b, h, w, d = zp.shape
    z_flat = zp.reshape(-1, d)
    idx3, dmin3 = _dist_argmin(z_flat, weight)
    vq_indices = idx3.reshape(-1)
    z_quantized = _sc_gather(weight, vq_indices)
    commitment_loss = jnp.sum(dmin3) / jnp.float32(TOKENS * DIM)
    loss = 0.25 * commitment_loss
    q = jnp.transpose(z_quantized.reshape(b, h, w, d), (0, 3, 1, 2))
    return (q, loss, commitment_loss)


# speedup vs baseline: 1.4528x; 1.2246x over previous
import functools

import jax
import jax.numpy as jnp
from jax import lax
from jax.experimental import pallas as pl
from jax.experimental.pallas import tpu as pltpu
from jax.experimental.pallas import tpu_sc as plsc

NUM_CODES = 8192
DIM = 256
TOKENS = 8192
BM = 512
NB = TOKENS // BM

_SC_INFO = plsc.get_sparse_core_info()
_NC, _NS = _SC_INFO.num_cores, _SC_INFO.num_subcores
_NW = _NC * _NS
_BPW = TOKENS // _NW


def _dist_argmin_kernel(z_ref, w_ref, idx_ref, dmin_ref, csq_ref):
    @pl.when(pl.program_id(0) == 0)
    def _():
        w = w_ref[...]
        csq_ref[0, :] = jnp.sum(w * w, axis=1)

    z = z_ref[...]
    zsq = jnp.sum(z * z, axis=1, keepdims=True)
    mm2 = lax.dot_general(z + z, w_ref[...], (((1,), (1,)), ((), ())),
                          preferred_element_type=jnp.float32)
    CH = 128
    NCH = NUM_CODES // CH
    m = (zsq + csq_ref[0, 0:CH][None, :]) - mm2[:, 0:CH]
    c = jnp.zeros((BM, CH), jnp.int32)
    for k in range(1, NCH):
        v = (zsq + csq_ref[0, k * CH:(k + 1) * CH][None, :]) - mm2[:, k * CH:(k + 1) * CH]
        upd = v < m
        m = jnp.where(upd, v, m)
        c = jnp.where(upd, k, c)
    jj = c * CH + lax.broadcasted_iota(jnp.int32, (BM, CH), 1)
    mn = jnp.min(m, axis=1, keepdims=True)
    idx = jnp.min(jnp.where(m == mn, jj, jnp.int32(2**30)), axis=1)
    idx_ref[0, 0, :] = idx
    dmin_ref[0, 0, :] = mn[:, 0]


def _dist_argmin(z_flat, weight):
    return pl.pallas_call(
        _dist_argmin_kernel,
        grid=(NB,),
        in_specs=[
            pl.BlockSpec((BM, DIM), lambda i: (i, 0)),
            pl.BlockSpec((NUM_CODES, DIM), lambda i: (0, 0)),
        ],
        out_specs=[
            pl.BlockSpec((1, 1, BM), lambda i: (i, 0, 0)),
            pl.BlockSpec((1, 1, BM), lambda i: (i, 0, 0)),
        ],
        out_shape=[
            jax.ShapeDtypeStruct((NB, 1, BM), jnp.int32),
            jax.ShapeDtypeStruct((NB, 1, BM), jnp.float32),
        ],
        scratch_shapes=[pltpu.VMEM((1, NUM_CODES), jnp.float32)],
        compiler_params=pltpu.CompilerParams(
            dimension_semantics=("arbitrary",),
        ),
    )(z_flat, weight)


@functools.partial(
    pl.kernel,
    mesh=plsc.VectorSubcoreMesh(core_axis_name="c", subcore_axis_name="s"),
    out_type=jax.ShapeDtypeStruct((TOKENS, DIM), jnp.float32),
    scratch_types=[
        pltpu.VMEM((_BPW,), jnp.int32),
        pltpu.VMEM((_BPW, DIM), jnp.float32),
        pltpu.SemaphoreType.DMA,
    ],
)
def _sc_gather(table_hbm, idx_hbm, out_hbm, idx_v, rows_v, sem):
    wid = lax.axis_index("s") * _NC + lax.axis_index("c")
    base = wid * _BPW
    pltpu.sync_copy(idx_hbm.at[pl.ds(base, _BPW)], idx_v)
    pltpu.async_copy(table_hbm.at[idx_v], rows_v, sem).wait()
    pltpu.sync_copy(rows_v, out_hbm.at[pl.ds(base, _BPW)])


def kernel(z, weight):
    zp = jnp.transpose(z, (0, 2, 3, 1))
    b, h, w, d = zp.shape
    z_flat = zp.reshape(-1, d)
    idx3, dmin3 = _dist_argmin(z_flat, weight)
    vq_indices = idx3.reshape(-1)
    z_quantized = _sc_gather(weight, vq_indices)
    commitment_loss = jnp.sum(dmin3) / jnp.float32(TOKENS * DIM)
    loss = 0.25 * commitment_loss
    q = jnp.transpose(z_quantized.reshape(b, h, w, d), (0, 3, 1, 2))
    return (q, loss, commitment_loss)
